# baseline (speedup 1.0000x reference)
import functools

import jax
import jax.numpy as jnp
from jax.experimental import pallas as pl
from jax.experimental.pallas import tpu as pltpu


def _pool_linear_kernel(inv_hw, x_ref, w_ref, b_ref, o_ref):
    pooled = jnp.sum(x_ref[...], axis=0) * inv_hw
    o_ref[...] = (jnp.dot(pooled, w_ref[...],
                          preferred_element_type=jnp.float32)
                  + b_ref[...]).astype(o_ref.dtype)


def _choose_tn(n):
    for t in (64, 32, 16, 8):
        if n % t == 0 and n // t >= 2:
            return t
    return n


def kernel(x, w_f, b_f):
    n, c, h, w = x.shape
    hw = h * w
    out_dim = w_f.shape[1]
    xt = jnp.transpose(x, (2, 3, 0, 1)).reshape(hw, n, c)
    tn = _choose_tn(n)
    body = functools.partial(_pool_linear_kernel, 1.0 / float(hw))
    return pl.pallas_call(
        body,
        out_shape=jax.ShapeDtypeStruct((n, out_dim), jnp.float32),
        grid=(pl.cdiv(n, tn),),
        in_specs=[
            pl.BlockSpec((hw, tn, c), lambda i: (0, i, 0)),
            pl.BlockSpec(w_f.shape, lambda i: (0, 0)),
            pl.BlockSpec(b_f.shape, lambda i: (0, 0)),
        ],
        out_specs=pl.BlockSpec((tn, out_dim), lambda i: (i, 0)),
        compiler_params=pltpu.CompilerParams(
            dimension_semantics=("parallel",)),
        cost_estimate=pl.CostEstimate(
            flops=int(n * c * hw + 2 * n * c * out_dim),
            transcendentals=0,
            bytes_accessed=int(x.size * x.dtype.itemsize
                               + (w_f.size + b_f.size) * 4
                               + n * out_dim * 4),
        ),
    )(xt, w_f, b_f.astype(jnp.float32))


# speedup vs baseline: 1.3333x; 1.3333x over previous
import functools

import jax
import jax.numpy as jnp
from jax.experimental import pallas as pl
from jax.experimental.pallas import tpu as pltpu


def _pool_linear_kernel(inv_hw, x_ref, wt_ref, bt_ref, o_ref):
    pooled = jnp.sum(x_ref[...], axis=0) * inv_hw
    ot = jax.lax.dot_general(wt_ref[...], pooled,
                             (((1,), (1,)), ((), ())),
                             preferred_element_type=jnp.float32)
    o_ref[...] = (ot + bt_ref[...]).astype(o_ref.dtype)


def _choose_tn(n):
    for t in (128, 64, 32, 16, 8):
        if n % t == 0 and n // t >= 2:
            return t
    return n


def kernel(x, w_f, b_f):
    n, c, h, w = x.shape
    hw = h * w
    out_dim = w_f.shape[1]
    xt = jnp.transpose(x, (2, 3, 0, 1)).reshape(hw, n, c)
    wt = w_f.T
    bt = jnp.transpose(b_f)
    tn = _choose_tn(n)
    body = functools.partial(_pool_linear_kernel, 1.0 / float(hw))
    ot = pl.pallas_call(
        body,
        out_shape=jax.ShapeDtypeStruct((out_dim, n), jnp.float32),
        grid=(pl.cdiv(n, tn),),
        in_specs=[
            pl.BlockSpec((hw, tn, c), lambda i: (0, i, 0)),
            pl.BlockSpec((out_dim, c), lambda i: (0, 0)),
            pl.BlockSpec((out_dim, 1), lambda i: (0, 0)),
        ],
        out_specs=pl.BlockSpec((out_dim, tn), lambda i: (0, i)),
        compiler_params=pltpu.CompilerParams(
            dimension_semantics=("parallel",)),
        cost_estimate=pl.CostEstimate(
            flops=int(n * c * hw + 2 * n * c * out_dim),
            transcendentals=0,
            bytes_accessed=int(x.size * x.dtype.itemsize
                               + (w_f.size + b_f.size) * 4
                               + n * out_dim * 4),
        ),
    )(xt, wt, bt.astype(jnp.float32))
    return ot.T
